# Initial kernel scaffold; baseline (speedup 1.0000x reference)
import jax
import jax.numpy as jnp
from jax.experimental import pallas as pl


def kernel(adj_indices, adj_values, features, W, b, gamma, beta):
    raise NotImplementedError("write your pallas kernel here")


# speedup vs baseline: 4.4936x; 4.4936x over previous
import functools

import jax
import jax.numpy as jnp
from jax import lax
from jax.experimental import pallas as pl
from jax.experimental.pallas import tpu as pltpu
from jax.experimental.pallas import tpu_sc as plsc

N = 10000
E = 320000
D = 128

NC = 2
NS = 16
NW = NC * NS
EW = E // NW
CHUNK = 80
NCHUNK = EW // CHUNK
RPT = 624
ZR = 208


def _mm_body(x_ref, w_ref, o_ref):
    o_ref[...] = jnp.dot(x_ref[...], w_ref[...],
                         preferred_element_type=jnp.float32)


def _tc_matmul(x, w):
    bm = 1000
    return pl.pallas_call(
        _mm_body,
        grid=(N // bm,),
        in_specs=[
            pl.BlockSpec((bm, D), lambda i: (i, 0)),
            pl.BlockSpec((D, D), lambda i: (0, 0)),
        ],
        out_specs=pl.BlockSpec((bm, D), lambda i: (i, 0)),
        out_shape=jax.ShapeDtypeStruct((N, D), jnp.float32),
    )(x, w)


def _sc_spmm_body(base_hbm, row_hbm, col_hbm, val_hbm, out_hbm,
                  rowi_v, coli_v, val_v, rows_v, zbuf_v, acc_sh, sem):
    cid = lax.axis_index("c")
    sid = lax.axis_index("s")
    wid = cid * NS + sid

    for jj in range(8):
        zbuf_v[0, pl.ds(jj * 16, 16)] = jnp.zeros((16,), jnp.float32)

    def zrow_body(i, _):
        for jj in range(8):
            sl = pl.ds(jj * 16, 16)
            zbuf_v[i, sl] = zbuf_v[0, sl]
        return _

    lax.fori_loop(1, ZR, zrow_body, 0)
    r0 = sid * RPT
    for k in range(RPT // ZR):
        pltpu.sync_copy(zbuf_v, acc_sh.at[pl.ds(r0 + k * ZR, ZR), :])

    @pl.when(sid == NS - 1)
    def _():
        pltpu.sync_copy(zbuf_v.at[pl.ds(0, 16), :],
                        acc_sh.at[pl.ds(NS * RPT, 16), :])

    plsc.subcore_barrier()

    ebase = wid * EW

    def chunk_body(j, _):
        off = ebase + j * CHUNK
        pltpu.sync_copy(row_hbm.at[pl.ds(off, CHUNK)], rowi_v)
        pltpu.sync_copy(col_hbm.at[pl.ds(off, CHUNK)], coli_v)
        pltpu.sync_copy(val_hbm.at[pl.ds(off, CHUNK)], val_v)
        pltpu.async_copy(base_hbm.at[coli_v], rows_v, sem).wait()

        def grp_body(g, _):
            vsl = val_v[pl.ds(g * 16, 16)]
            for lane in range(16):
                vb = lax.gather(
                    vsl, jnp.full((16, 1), lane, jnp.int32),
                    lax.GatherDimensionNumbers(
                        offset_dims=(), collapsed_slice_dims=(0,),
                        start_index_map=(0,)),
                    (1,), mode=lax.GatherScatterMode.PROMISE_IN_BOUNDS)
                e = g * 16 + lane
                for jj in range(8):
                    sl = pl.ds(jj * 16, 16)
                    rows_v[e, sl] = rows_v[e, sl] * vb
            return _

        lax.fori_loop(0, CHUNK // 16, grp_body, 0)
        pltpu.sync_copy(rows_v, acc_sh.at[rowi_v], add=True)
        return _

    lax.fori_loop(0, NCHUNK, chunk_body, 0)

    plsc.subcore_barrier()
    pltpu.sync_copy(acc_sh.at[pl.ds(r0, RPT), :],
                    out_hbm.at[cid, pl.ds(r0, RPT), :])

    @pl.when(sid == NS - 1)
    def _():
        pltpu.sync_copy(acc_sh.at[pl.ds(NS * RPT, 16), :],
                        out_hbm.at[cid, pl.ds(NS * RPT, 16), :])


def _sc_spmm(base, row, col, val):
    mesh = plsc.VectorSubcoreMesh(core_axis_name="c", subcore_axis_name="s")
    f = pl.kernel(
        _sc_spmm_body,
        out_type=jax.ShapeDtypeStruct((NC, N, D), jnp.float32),
        mesh=mesh,
        scratch_types=[
            pltpu.VMEM((CHUNK,), jnp.int32),
            pltpu.VMEM((CHUNK,), jnp.int32),
            pltpu.VMEM((CHUNK,), jnp.float32),
            pltpu.VMEM((CHUNK, D), jnp.float32),
            pltpu.VMEM((ZR, D), jnp.float32),
            pltpu.VMEM_SHARED((N, D), jnp.float32),
            pltpu.SemaphoreType.DMA,
        ],
    )
    return f(base, row, col, val)


def _fin_body(p_ref, b_ref, g_ref, bt_ref, o_ref):
    h = p_ref[0] + p_ref[1] + b_ref[...]
    h = jnp.where(h > 0, h, jnp.exp(jnp.minimum(h, 0.0)) - 1.0)
    mean = jnp.mean(h, axis=-1, keepdims=True)
    var = jnp.mean((h - mean) * (h - mean), axis=-1, keepdims=True)
    o_ref[...] = (h - mean) / jnp.sqrt(var + 1e-5) * g_ref[...] + bt_ref[...]


def _tc_finish(partials, b, gamma, beta):
    bm = 1000
    return pl.pallas_call(
        _fin_body,
        grid=(N // bm,),
        in_specs=[
            pl.BlockSpec((NC, bm, D), lambda i: (0, i, 0)),
            pl.BlockSpec((1, D), lambda i: (0, 0)),
            pl.BlockSpec((1, D), lambda i: (0, 0)),
            pl.BlockSpec((1, D), lambda i: (0, 0)),
        ],
        out_specs=pl.BlockSpec((bm, D), lambda i: (i, 0)),
        out_shape=jax.ShapeDtypeStruct((N, D), jnp.float32),
    )(partials, b, gamma, beta)


@jax.jit
def kernel(adj_indices, adj_values, features, W, b, gamma, beta):
    base = _tc_matmul(features, W)
    row = adj_indices[0]
    col = adj_indices[1]
    partials = _sc_spmm(base, row, col, adj_values)
    return _tc_finish(partials, b,
                      gamma.reshape(1, D), beta.reshape(1, D))
